# baseline (speedup 1.0000x reference)
import functools

import jax
import jax.numpy as jnp
from jax import lax
from jax.experimental import pallas as pl
from jax.experimental.pallas import tpu as pltpu
from jax.experimental.pallas import tpu_sc as plsc

NE = 1000000
ED = 64
B = 4096
L = 200

_NC = 2
_NS = 16
_NW = _NC * _NS
_BPW = B // _NW
_BLK = ED * L
_L_OFFS = tuple(k * 16 for k in range(12)) + (L - 16,)
_G0, _G1 = 104, 96


def _emb_body(inp_hbm, table_hbm, out_hbm, idx_v, rows_v, outt_v, sem):
    c = lax.axis_index("c")
    s = lax.axis_index("s")
    wid = s * _NC + c
    iota_l = lax.iota(jnp.int32, 16) * L

    def do_b(i, carry):
        b = wid * _BPW + i
        pltpu.sync_copy(inp_hbm.at[b], idx_v)
        for off in _L_OFFS:
            v = idx_v[pl.ds(off, 16)]
            idx_v[pl.ds(off, 16)] = jnp.clip(v, 0, NE - 1)
        cp1 = pltpu.async_copy(
            table_hbm.at[idx_v.at[pl.ds(0, _G0)]], rows_v.at[pl.ds(0, _G0)],
            sem)
        cp2 = pltpu.async_copy(
            table_hbm.at[idx_v.at[pl.ds(_G0, _G1)]],
            rows_v.at[pl.ds(_G0, _G1)], sem)
        cp1.wait()
        cp2.wait()

        def do_l(l, lcarry):
            for db in range(ED // 16):
                vals = rows_v[l, pl.ds(db * 16, 16)]
                plsc.store_scatter(outt_v, [iota_l + (db * 16 * L + l)], vals)
            return lcarry

        lax.fori_loop(0, L, do_l, 0)
        pltpu.sync_copy(outt_v, out_hbm.at[pl.ds(b * _BLK, _BLK)])
        return carry

    lax.fori_loop(0, _BPW, do_b, 0)


@functools.partial(jax.jit, static_argnums=())
def _emb_encoder(inp, table):
    mesh = plsc.VectorSubcoreMesh(core_axis_name="c", subcore_axis_name="s")
    out = pl.kernel(
        _emb_body,
        mesh=mesh,
        compiler_params=pltpu.CompilerParams(
            needs_layout_passes=False, use_tc_tiling_on_sc=False),
        out_type=jax.ShapeDtypeStruct((B * _BLK,), jnp.float32),
        scratch_types=[
            pltpu.VMEM((L,), jnp.int32),
            pltpu.VMEM((L, ED), jnp.float32),
            pltpu.VMEM((_BLK,), jnp.float32),
            pltpu.SemaphoreType.DMA,
        ],
    )(inp, table)
    return out


def kernel(input, emb_weight):
    inp = input.astype(jnp.int32)
    out = _emb_encoder(inp, emb_weight)
    return out.reshape(B, ED, L)


# speedup vs baseline: 1.3820x; 1.3820x over previous
import functools

import jax
import jax.numpy as jnp
from jax import lax
from jax.experimental import pallas as pl
from jax.experimental.pallas import tpu as pltpu
from jax.experimental.pallas import tpu_sc as plsc

NE = 1000000
ED = 64
B = 4096
L = 200

_NC = 2
_NS = 16
_NW = _NC * _NS
_BPW = B // _NW
_BLK = ED * L
_IDXW = _BPW * L
_G0, _G1 = 104, 96


def _emb_body(inp_hbm, table_hbm, out_hbm,
              idx_v, rows_a, rows_b, out_a, out_b,
              sem_ga, sem_gb, sem_wa, sem_wb):
    c = lax.axis_index("c")
    s = lax.axis_index("s")
    wid = s * _NC + c
    b_base = wid * _BPW
    iota_l = lax.iota(jnp.int32, 16) * L

    pltpu.sync_copy(inp_hbm.at[pl.ds(wid * _IDXW, _IDXW)], idx_v)

    @plsc.parallel_loop(0, _IDXW, 16, unroll=8)
    def _clamp(i):
        idx_v[pl.ds(i, 16)] = jnp.clip(idx_v[pl.ds(i, 16)], 0, NE - 1)

    def issue_gather(rows_ref, sem, i):
        pltpu.async_copy(
            table_hbm.at[idx_v.at[pl.ds(i * L, _G0)]],
            rows_ref.at[pl.ds(0, _G0)], sem)
        pltpu.async_copy(
            table_hbm.at[idx_v.at[pl.ds(i * L + _G0, _G1)]],
            rows_ref.at[pl.ds(_G0, _G1)], sem)

    def wait_gather(rows_ref, sem):
        pltpu.make_async_copy(table_hbm.at[pl.ds(0, L)], rows_ref, sem).wait()

    def transpose(rows_ref, out_ref):
        @plsc.parallel_loop(0, L, 1, unroll=8)
        def _t(l):
            for db in range(ED // 16):
                vals = rows_ref[l, pl.ds(db * 16, 16)]
                plsc.store_scatter(
                    out_ref, [iota_l + (db * 16 * L + l)], vals)

    def issue_write(out_ref, sem, b):
        pltpu.async_copy(out_ref, out_hbm.at[pl.ds(b * _BLK, _BLK)], sem)

    def wait_write(out_ref, sem, b):
        pltpu.make_async_copy(
            out_ref, out_hbm.at[pl.ds(b * _BLK, _BLK)], sem).wait()

    issue_gather(rows_a, sem_ga, 0)

    def do_pair(p, carry):
        i0 = 2 * p
        b0 = b_base + i0
        issue_gather(rows_b, sem_gb, i0 + 1)
        wait_gather(rows_a, sem_ga)

        @pl.when(p > 0)
        def _():
            wait_write(out_a, sem_wa, b0 - 2)

        transpose(rows_a, out_a)
        issue_write(out_a, sem_wa, b0)

        @pl.when(p < _BPW // 2 - 1)
        def _():
            issue_gather(rows_a, sem_ga, i0 + 2)

        wait_gather(rows_b, sem_gb)

        @pl.when(p > 0)
        def _():
            wait_write(out_b, sem_wb, b0 - 1)

        transpose(rows_b, out_b)
        issue_write(out_b, sem_wb, b0 + 1)
        return carry

    lax.fori_loop(0, _BPW // 2, do_pair, 0)
    wait_write(out_a, sem_wa, b_base + _BPW - 2)
    wait_write(out_b, sem_wb, b_base + _BPW - 1)


@jax.jit
def _emb_encoder(inp, table):
    mesh = plsc.VectorSubcoreMesh(core_axis_name="c", subcore_axis_name="s")
    out = pl.kernel(
        _emb_body,
        mesh=mesh,
        compiler_params=pltpu.CompilerParams(
            needs_layout_passes=False, use_tc_tiling_on_sc=False),
        out_type=jax.ShapeDtypeStruct((B * _BLK,), jnp.float32),
        scratch_types=[
            pltpu.VMEM((_IDXW,), jnp.int32),
            pltpu.VMEM((L, ED), jnp.float32),
            pltpu.VMEM((L, ED), jnp.float32),
            pltpu.VMEM((_BLK,), jnp.float32),
            pltpu.VMEM((_BLK,), jnp.float32),
            pltpu.SemaphoreType.DMA,
            pltpu.SemaphoreType.DMA,
            pltpu.SemaphoreType.DMA,
            pltpu.SemaphoreType.DMA,
        ],
    )(inp, table)
    return out


def kernel(input, emb_weight):
    inp = input.astype(jnp.int32).reshape(B * L)
    out = _emb_encoder(inp, emb_weight)
    return out.reshape(B, ED, L)
